# baseline (speedup 1.0000x reference)
import functools

import jax
import jax.numpy as jnp
from jax import lax
from jax.experimental import pallas as pl
from jax.experimental.pallas import tpu as pltpu
from jax.experimental.pallas import tpu_sc as plsc

N = 10000
E = 320000
H = 128
NC, NS, L = 2, 16, 16
NW = NC * NS
CH = 128
CPW = 80
EP = NW * CPW * CH
ND = 10240
TN = 2048
NT = ND // TN
TE = 4096
PREC = lax.Precision.HIGHEST

f32 = jnp.float32


def _dot(a, b):
    return lax.dot_general(a, b, (((a.ndim - 1,), (0,)), ((), ())),
                           preferred_element_type=f32, precision=PREC)


def _dot_t(a, b):
    return lax.dot_general(a, b, (((0,), (0,)), ((), ())),
                           preferred_element_type=f32, precision=PREC)


def _enc_body(s_ref, w0, b0, w1, b1, w2, b2, out_ref):
    h = jnp.maximum(_dot(s_ref[...], w0[...]) + b0[...], 0.0)
    h = jnp.maximum(_dot(h, w1[...]) + b1[...], 0.0)
    out_ref[...] = _dot(h, w2[...]) + b2[...]


def _front_body(xa_ref, npm_ref, emb_ref, out_ref, m_ref, z_ref):
    ph = pl.program_id(0)
    t = pl.program_id(1)
    logits = _dot(xa_ref[...], npm_ref[...])

    @pl.when(ph == 0)
    def _():
        tm = jnp.max(logits, axis=1, keepdims=True)

        @pl.when(t == 0)
        def _():
            m_ref[...] = tm
            z_ref[...] = jnp.sum(jnp.exp(logits - tm), axis=1, keepdims=True)

        @pl.when(t > 0)
        def _():
            m_old = m_ref[...]
            m_new = jnp.maximum(m_old, tm)
            z_ref[...] = (z_ref[...] * jnp.exp(m_old - m_new)
                          + jnp.sum(jnp.exp(logits - m_new), axis=1, keepdims=True))
            m_ref[...] = m_new

    @pl.when(ph == 1)
    def _():
        w = jnp.exp(logits - m_ref[...]) / z_ref[...]
        out_ref[...] = _dot_t(w, emb_ref[...])


def _ab_body(lat_ref, w1a, w1b, b1, a_ref, b_ref):
    latv = lat_ref[...]
    a_ref[...] = _dot(latv, w1a[...]) + b1[...]
    b_ref[...] = _dot(latv, w1b[...])


def _h2_body(g_ref, w2, b2, out_ref):
    out_ref[...] = jnp.maximum(_dot(g_ref[...], w2[...]) + b2[...], 0.0)


def _node_body(s2_ref, c2_ref, lat_ref, w3, b3, nw1a, nw1b, nb1, nw2, nb2,
               nw3, nb3, out_ref):
    s_sum = s2_ref[0] + s2_ref[1]
    counts = c2_ref[0] + c2_ref[1]
    inbox = _dot(s_sum, w3[...]) + counts * b3[...]
    latv = lat_ref[...]
    u = jnp.maximum(_dot(latv, nw1a[...]) + _dot(inbox, nw1b[...]) + nb1[...], 0.0)
    u = jnp.maximum(_dot(u, nw2[...]) + nb2[...], 0.0)
    out_ref[...] = latv + _dot(u, nw3[...]) + nb3[...]


def _back_body(qa_ref, npm_ref, lat_ref, d1z, d1q, db1, d2w, db2, d3w, db3,
               out_ref, m_ref, z_ref, zacc_ref):
    ph = pl.program_id(0)
    t = pl.program_id(1)
    logits = _dot(qa_ref[...], npm_ref[...])

    @pl.when(ph == 0)
    def _():
        tm = jnp.max(logits, axis=1, keepdims=True)

        @pl.when(t == 0)
        def _():
            m_ref[...] = tm
            z_ref[...] = jnp.sum(jnp.exp(logits - tm), axis=1, keepdims=True)

        @pl.when(t > 0)
        def _():
            m_old = m_ref[...]
            m_new = jnp.maximum(m_old, tm)
            z_ref[...] = (z_ref[...] * jnp.exp(m_old - m_new)
                          + jnp.sum(jnp.exp(logits - m_new), axis=1, keepdims=True))
            m_ref[...] = m_new

    @pl.when(ph == 1)
    def _():
        w = jnp.exp(logits - m_ref[...]) / z_ref[...]
        contrib = _dot(w, lat_ref[...])

        @pl.when(t == 0)
        def _():
            zacc_ref[...] = contrib

        @pl.when(t > 0)
        def _():
            zacc_ref[...] = zacc_ref[...] + contrib

        @pl.when(t == NT - 1)
        def _():
            z = zacc_ref[...]
            h = jnp.maximum(_dot(z, d1z[...]) + _dot(qa_ref[...], d1q[...])
                            + db1[...], 0.0)
            h = jnp.maximum(_dot(h, d2w[...]) + db2[...], 0.0)
            out_ref[...] = _dot(h, d3w[...]) + db3[...]


X0 = 54
NPAIR = 160


def _gather_body(a_hbm, b_hbm, ridx_hbm, sidx_hbm, g_hbm,
                 idxr_v, idxs_v, ra0, rb0, ra1, rb1, sa0, sb0, sa1, sb1):
    cid = lax.axis_index("c")
    sid = lax.axis_index("s")
    base_pair = sid * NPAIR
    pltpu.sync_copy(ridx_hbm.at[pl.ds(base_pair, NPAIR)], idxr_v)
    pltpu.sync_copy(sidx_hbm.at[pl.ds(base_pair, NPAIR)], idxs_v)

    loc0 = cid * X0
    count = jnp.where(cid == 0, X0, NPAIR - X0)

    ras, rbs = (ra0, ra1), (rb0, rb1)
    sas, sbs = (sa0, sa1), (sb0, sb1)

    def start(c_local, slot):
        pltpu.async_copy(a_hbm.at[idxr_v.at[loc0 + c_local]], ras[slot],
                         sas[slot])
        pltpu.async_copy(b_hbm.at[idxs_v.at[loc0 + c_local]], rbs[slot],
                         sbs[slot])

    start(0, 0)

    def pair(j, carry):
        for b in range(2):
            cur = 2 * j + b
            nxt = cur + 1
            slot = b
            other = 1 - b

            @pl.when(nxt < count)
            def _():
                start(nxt, other)

            pltpu.make_async_copy(a_hbm.at[idxr_v.at[loc0 + cur]], ras[slot],
                                  sas[slot]).wait()
            pltpu.make_async_copy(b_hbm.at[idxs_v.at[loc0 + cur]], rbs[slot],
                                  sbs[slot]).wait()
            ra, rb = ras[slot], rbs[slot]

            def row(i, c2):
                for c in range(H // L):
                    sl = pl.ds(c * L, L)
                    ra[i, sl] = jnp.maximum(ra[i, sl] + rb[i, sl], 0.0)
                return c2

            lax.fori_loop(0, CH, row, 0)
            pltpu.sync_copy(ra,
                            g_hbm.at[pl.ds((base_pair + loc0 + cur) * CH, CH)])
        return carry

    lax.fori_loop(0, count // 2, pair, 0)


def _scatter_body(h2_hbm, ridx_hbm, zeros_hbm, s2_out, s_sh,
                  idx_v, r0, r1, s0, s1):
    cid = lax.axis_index("c")
    sid = lax.axis_index("s")
    wid = sid * NC + cid
    c0 = wid * CPW
    rows_per_tile = ND // NS
    lo = sid * rows_per_tile

    pltpu.sync_copy(ridx_hbm.at[pl.ds(c0, CPW)], idx_v)
    pltpu.sync_copy(zeros_hbm.at[pl.ds(lo, rows_per_tile)],
                    s_sh.at[pl.ds(lo, rows_per_tile)])
    plsc.subcore_barrier()

    rs = (r0, r1)
    ss = (s0, s1)

    def start(c_local, slot):
        pltpu.async_copy(h2_hbm.at[pl.ds((c0 + c_local) * CH, CH)],
                         rs[slot], ss[slot])

    start(0, 0)

    def pair(j, carry):
        for b in range(2):
            cur = j + b
            nxt = cur + 1
            slot = b
            other = 1 - b

            @pl.when(nxt < CPW)
            def _():
                start(nxt, other)

            pltpu.make_async_copy(h2_hbm.at[pl.ds((c0 + cur) * CH, CH)],
                                  rs[slot], ss[slot]).wait()
            pltpu.sync_copy(rs[slot], s_sh.at[idx_v.at[cur]], add=True)
        return carry

    lax.fori_loop(0, CPW // 2, lambda jj, c: pair(jj * 2, c), 0)
    plsc.subcore_barrier()

    pltpu.sync_copy(s_sh.at[pl.ds(lo, rows_per_tile)],
                    s2_out.at[cid, pl.ds(lo, rows_per_tile)])


def _row(b):
    return b.reshape(1, -1).astype(f32)


def kernel(x, s, q, node_pos, senders, receivers, params):
    P = x.shape[1]
    Q = q.shape[1]
    x2, s2, q2 = x[0].astype(f32), s[0].astype(f32), q[0].astype(f32)
    np32 = node_pos.astype(f32)
    r32 = receivers.astype(jnp.int32)
    s32 = senders.astype(jnp.int32)

    npm = jnp.concatenate([2.0 * np32.T,
                           -jnp.sum(np32 * np32, axis=1)[None, :],
                           jnp.zeros((4, N), f32)], axis=0)
    npm_pad = jnp.zeros((8, ND - N), f32).at[3, :].set(-1e30)
    npm = jnp.concatenate([npm, npm_pad], axis=1)
    xa = jnp.concatenate([x2, jnp.ones((P, 1), f32), jnp.zeros((P, 4), f32)], axis=1)
    qa = jnp.concatenate([q2, jnp.ones((Q, 1), f32), jnp.zeros((Q, 4), f32)], axis=1)

    pad = EP - E
    r_g = jnp.concatenate([r32, jnp.zeros((pad,), jnp.int32)]).reshape(EP // CH, CH)
    s_g = jnp.concatenate([s32, jnp.zeros((pad,), jnp.int32)]).reshape(EP // CH, CH)
    r_sc = jnp.concatenate([r32, jnp.full((pad,), N, jnp.int32)]).reshape(EP // CH, CH)

    zeros_nd = jnp.zeros((ND, H), f32)

    enc = params["enc"]
    dec = params["dec"]

    emb = pl.pallas_call(
        _enc_body,
        out_shape=jax.ShapeDtypeStruct((P, H), f32),
    )(s2, enc[0][0], _row(enc[0][1]), enc[1][0], _row(enc[1][1]),
      enc[2][0], _row(enc[2][1]))

    latents = pl.pallas_call(
        _front_body,
        grid=(2, NT),
        in_specs=[
            pl.BlockSpec((P, 8), lambda ph, t: (0, 0)),
            pl.BlockSpec((8, TN), lambda ph, t: (0, t)),
            pl.BlockSpec((P, H), lambda ph, t: (0, 0)),
        ],
        out_specs=pl.BlockSpec((TN, H), lambda ph, t: (t, 0)),
        out_shape=jax.ShapeDtypeStruct((ND, H), f32),
        scratch_shapes=[pltpu.VMEM((P, 1), f32), pltpu.VMEM((P, 1), f32)],
    )(xa, npm, emb)

    mesh = plsc.VectorSubcoreMesh(core_axis_name="c", subcore_axis_name="s",
                                  num_cores=NC, num_subcores=NS)

    gather_call = pl.kernel(
        _gather_body,
        out_type=jax.ShapeDtypeStruct((EP, H), f32),
        mesh=mesh,
        scratch_types=[
            pltpu.VMEM((NPAIR, CH), jnp.int32),
            pltpu.VMEM((NPAIR, CH), jnp.int32),
            pltpu.VMEM((CH, H), f32),
            pltpu.VMEM((CH, H), f32),
            pltpu.VMEM((CH, H), f32),
            pltpu.VMEM((CH, H), f32),
            pltpu.SemaphoreType.DMA,
            pltpu.SemaphoreType.DMA,
            pltpu.SemaphoreType.DMA,
            pltpu.SemaphoreType.DMA,
        ],
    )

    scatter_call = pl.kernel(
        _scatter_body,
        out_type=jax.ShapeDtypeStruct((NC, ND, H), f32),
        mesh=mesh,
        scratch_types=[
            pltpu.VMEM_SHARED((ND, H), f32),
            pltpu.VMEM((CPW, CH), jnp.int32),
            pltpu.VMEM((CH, H), f32),
            pltpu.VMEM((CH, H), f32),
            pltpu.SemaphoreType.DMA,
            pltpu.SemaphoreType.DMA,
        ],
    )

    counts_parts = scatter_call(jnp.ones((EP, H), f32), r_sc, zeros_nd)
    counts2 = counts_parts[:, :, :1]
    for bi, bp in enumerate(params["blocks"]):
        (w1, b1), (w2, b2), (w3, b3) = bp["msg"]
        (nw1, nb1), (nw2, nb2), (nw3, nb3) = bp["node"]
        w1a, w1b = w1[:H], w1[H:]
        nw1a, nw1b = nw1[:H], nw1[H:]

        a_nodes, b_nodes = pl.pallas_call(
            _ab_body,
            grid=(NT,),
            in_specs=[
                pl.BlockSpec((TN, H), lambda t: (t, 0)),
                pl.BlockSpec((H, H), lambda t: (0, 0)),
                pl.BlockSpec((H, H), lambda t: (0, 0)),
                pl.BlockSpec((1, H), lambda t: (0, 0)),
            ],
            out_specs=[pl.BlockSpec((TN, H), lambda t: (t, 0)),
                       pl.BlockSpec((TN, H), lambda t: (t, 0))],
            out_shape=[jax.ShapeDtypeStruct((ND, H), f32),
                       jax.ShapeDtypeStruct((ND, H), f32)],
        )(latents, w1a, w1b, _row(b1))

        g_edges = gather_call(a_nodes, b_nodes, r_g, s_g)

        h2 = pl.pallas_call(
            _h2_body,
            grid=(EP // TE,),
            in_specs=[
                pl.BlockSpec((TE, H), lambda t: (t, 0)),
                pl.BlockSpec((H, H), lambda t: (0, 0)),
                pl.BlockSpec((1, H), lambda t: (0, 0)),
            ],
            out_specs=pl.BlockSpec((TE, H), lambda t: (t, 0)),
            out_shape=jax.ShapeDtypeStruct((EP, H), f32),
        )(g_edges, w2, _row(b2))

        s2_parts = scatter_call(h2, r_sc, zeros_nd)

        latents = pl.pallas_call(
            _node_body,
            grid=(NT,),
            in_specs=[
                pl.BlockSpec((NC, TN, H), lambda t: (0, t, 0)),
                pl.BlockSpec((NC, TN, 1), lambda t: (0, t, 0)),
                pl.BlockSpec((TN, H), lambda t: (t, 0)),
                pl.BlockSpec((H, H), lambda t: (0, 0)),
                pl.BlockSpec((1, H), lambda t: (0, 0)),
                pl.BlockSpec((H, H), lambda t: (0, 0)),
                pl.BlockSpec((H, H), lambda t: (0, 0)),
                pl.BlockSpec((1, H), lambda t: (0, 0)),
                pl.BlockSpec((H, H), lambda t: (0, 0)),
                pl.BlockSpec((1, H), lambda t: (0, 0)),
                pl.BlockSpec((H, H), lambda t: (0, 0)),
                pl.BlockSpec((1, H), lambda t: (0, 0)),
            ],
            out_specs=pl.BlockSpec((TN, H), lambda t: (t, 0)),
            out_shape=jax.ShapeDtypeStruct((ND, H), f32),
        )(s2_parts, counts2, latents,
          w3, _row(b3), nw1a, nw1b, _row(nb1), nw2, _row(nb2), nw3, _row(nb3))

    d1 = dec[0][0]
    d1z = d1[:H]
    d1q = jnp.concatenate([d1[H:], jnp.zeros((5, H), f32)], axis=0)

    out = pl.pallas_call(
        _back_body,
        grid=(2, NT),
        in_specs=[
            pl.BlockSpec((Q, 8), lambda ph, t: (0, 0)),
            pl.BlockSpec((8, TN), lambda ph, t: (0, t)),
            pl.BlockSpec((TN, H), lambda ph, t: (t, 0)),
            pl.BlockSpec((H, H), lambda ph, t: (0, 0)),
            pl.BlockSpec((8, H), lambda ph, t: (0, 0)),
            pl.BlockSpec((1, H), lambda ph, t: (0, 0)),
            pl.BlockSpec((H, H), lambda ph, t: (0, 0)),
            pl.BlockSpec((1, H), lambda ph, t: (0, 0)),
            pl.BlockSpec((H, 8), lambda ph, t: (0, 0)),
            pl.BlockSpec((1, 8), lambda ph, t: (0, 0)),
        ],
        out_specs=pl.BlockSpec((Q, 8), lambda ph, t: (0, 0)),
        out_shape=jax.ShapeDtypeStruct((Q, 8), f32),
        scratch_shapes=[pltpu.VMEM((Q, 1), f32), pltpu.VMEM((Q, 1), f32),
                        pltpu.VMEM((Q, H), f32)],
    )(qa, npm, latents, d1z, d1q, _row(dec[0][1]), dec[1][0], _row(dec[1][1]),
      dec[2][0], _row(dec[2][1]))

    return out[None]


# speedup vs baseline: 1.0351x; 1.0351x over previous
import functools

import jax
import jax.numpy as jnp
from jax import lax
from jax.experimental import pallas as pl
from jax.experimental.pallas import tpu as pltpu
from jax.experimental.pallas import tpu_sc as plsc

N = 10000
E = 320000
H = 128
NC, NS, L = 2, 16, 16
NW = NC * NS
CH = 128
CPW = 80
EP = NW * CPW * CH
ND = 10240
TN = 2048
NT = ND // TN
TE = 4096
PREC = lax.Precision.HIGHEST

f32 = jnp.float32


def _dot(a, b):
    return lax.dot_general(a, b, (((a.ndim - 1,), (0,)), ((), ())),
                           preferred_element_type=f32, precision=PREC)


def _dot_t(a, b):
    return lax.dot_general(a, b, (((0,), (0,)), ((), ())),
                           preferred_element_type=f32, precision=PREC)


def _enc_body(s_ref, w0, b0, w1, b1, w2, b2, out_ref):
    h = jnp.maximum(_dot(s_ref[...], w0[...]) + b0[...], 0.0)
    h = jnp.maximum(_dot(h, w1[...]) + b1[...], 0.0)
    out_ref[...] = _dot(h, w2[...]) + b2[...]


def _front_body(xa_ref, npm_ref, emb_ref, out_ref, m_ref, z_ref):
    ph = pl.program_id(0)
    t = pl.program_id(1)
    logits = _dot(xa_ref[...], npm_ref[...])

    @pl.when(ph == 0)
    def _():
        tm = jnp.max(logits, axis=1, keepdims=True)

        @pl.when(t == 0)
        def _():
            m_ref[...] = tm
            z_ref[...] = jnp.sum(jnp.exp(logits - tm), axis=1, keepdims=True)

        @pl.when(t > 0)
        def _():
            m_old = m_ref[...]
            m_new = jnp.maximum(m_old, tm)
            z_ref[...] = (z_ref[...] * jnp.exp(m_old - m_new)
                          + jnp.sum(jnp.exp(logits - m_new), axis=1, keepdims=True))
            m_ref[...] = m_new

    @pl.when(ph == 1)
    def _():
        w = jnp.exp(logits - m_ref[...]) / z_ref[...]
        out_ref[...] = _dot_t(w, emb_ref[...])


def _ab_body(lat_ref, w1a, w1b, b1, a_ref, b_ref):
    latv = lat_ref[...]
    a_ref[...] = _dot(latv, w1a[...]) + b1[...]
    b_ref[...] = _dot(latv, w1b[...])


def _h2_body(g_ref, w2, b2, out_ref):
    out_ref[...] = jnp.maximum(_dot(g_ref[...], w2[...]) + b2[...], 0.0)


def _node_body(s2_ref, c2_ref, lat_ref, w3, b3, nw1a, nw1b, nb1, nw2, nb2,
               nw3, nb3, out_ref):
    s_sum = s2_ref[0] + s2_ref[1]
    counts = c2_ref[0] + c2_ref[1]
    inbox = _dot(s_sum, w3[...]) + counts * b3[...]
    latv = lat_ref[...]
    u = jnp.maximum(_dot(latv, nw1a[...]) + _dot(inbox, nw1b[...]) + nb1[...], 0.0)
    u = jnp.maximum(_dot(u, nw2[...]) + nb2[...], 0.0)
    out_ref[...] = latv + _dot(u, nw3[...]) + nb3[...]


def _back_body(qa_ref, npm_ref, lat_ref, d1z, d1q, db1, d2w, db2, d3w, db3,
               out_ref, m_ref, z_ref, zacc_ref):
    ph = pl.program_id(0)
    t = pl.program_id(1)
    logits = _dot(qa_ref[...], npm_ref[...])

    @pl.when(ph == 0)
    def _():
        tm = jnp.max(logits, axis=1, keepdims=True)

        @pl.when(t == 0)
        def _():
            m_ref[...] = tm
            z_ref[...] = jnp.sum(jnp.exp(logits - tm), axis=1, keepdims=True)

        @pl.when(t > 0)
        def _():
            m_old = m_ref[...]
            m_new = jnp.maximum(m_old, tm)
            z_ref[...] = (z_ref[...] * jnp.exp(m_old - m_new)
                          + jnp.sum(jnp.exp(logits - m_new), axis=1, keepdims=True))
            m_ref[...] = m_new

    @pl.when(ph == 1)
    def _():
        w = jnp.exp(logits - m_ref[...]) / z_ref[...]
        contrib = _dot(w, lat_ref[...])

        @pl.when(t == 0)
        def _():
            zacc_ref[...] = contrib

        @pl.when(t > 0)
        def _():
            zacc_ref[...] = zacc_ref[...] + contrib

        @pl.when(t == NT - 1)
        def _():
            z = zacc_ref[...]
            h = jnp.maximum(_dot(z, d1z[...]) + _dot(qa_ref[...], d1q[...])
                            + db1[...], 0.0)
            h = jnp.maximum(_dot(h, d2w[...]) + db2[...], 0.0)
            out_ref[...] = _dot(h, d3w[...]) + db3[...]


X0 = 106
NPAIR = 160


def _gather_body(a_hbm, b_hbm, ridx_hbm, sidx_hbm, g_hbm,
                 idxr_v, idxs_v, ra0, rb0, ra1, rb1, sa0, sb0, sa1, sb1):
    cid = lax.axis_index("c")
    sid = lax.axis_index("s")
    base_pair = sid * NPAIR
    pltpu.sync_copy(ridx_hbm.at[pl.ds(base_pair, NPAIR)], idxr_v)
    pltpu.sync_copy(sidx_hbm.at[pl.ds(base_pair, NPAIR)], idxs_v)

    loc0 = cid * X0
    count = jnp.where(cid == 0, X0, NPAIR - X0)

    ras, rbs = (ra0, ra1), (rb0, rb1)
    sas, sbs = (sa0, sa1), (sb0, sb1)

    def start(c_local, slot):
        pltpu.async_copy(a_hbm.at[idxr_v.at[loc0 + c_local]], ras[slot],
                         sas[slot])
        pltpu.async_copy(b_hbm.at[idxs_v.at[loc0 + c_local]], rbs[slot],
                         sbs[slot])

    start(0, 0)

    def pair(j, carry):
        for b in range(2):
            cur = 2 * j + b
            nxt = cur + 1
            slot = b
            other = 1 - b

            @pl.when(nxt < count)
            def _():
                start(nxt, other)

            pltpu.make_async_copy(a_hbm.at[idxr_v.at[loc0 + cur]], ras[slot],
                                  sas[slot]).wait()
            pltpu.make_async_copy(b_hbm.at[idxs_v.at[loc0 + cur]], rbs[slot],
                                  sbs[slot]).wait()
            ra, rb = ras[slot], rbs[slot]

            def row(i, c2):
                for c in range(H // L):
                    sl = pl.ds(c * L, L)
                    ra[i, sl] = jnp.maximum(ra[i, sl] + rb[i, sl], 0.0)
                return c2

            lax.fori_loop(0, CH, row, 0)
            pltpu.sync_copy(ra,
                            g_hbm.at[pl.ds((base_pair + loc0 + cur) * CH, CH)])
        return carry

    lax.fori_loop(0, count // 2, pair, 0)


def _scatter_body(h2_hbm, ridx_hbm, zeros_hbm, s2_out, s_sh,
                  idx_v, r0, r1, s0, s1):
    cid = lax.axis_index("c")
    sid = lax.axis_index("s")
    wid = sid * NC + cid
    c0 = wid * CPW
    rows_per_tile = ND // NS
    lo = sid * rows_per_tile

    pltpu.sync_copy(ridx_hbm.at[pl.ds(c0, CPW)], idx_v)
    pltpu.sync_copy(zeros_hbm.at[pl.ds(lo, rows_per_tile)],
                    s_sh.at[pl.ds(lo, rows_per_tile)])
    plsc.subcore_barrier()

    rs = (r0, r1)
    ss = (s0, s1)

    def start(c_local, slot):
        pltpu.async_copy(h2_hbm.at[pl.ds((c0 + c_local) * CH, CH)],
                         rs[slot], ss[slot])

    start(0, 0)

    def pair(j, carry):
        for b in range(2):
            cur = j + b
            nxt = cur + 1
            slot = b
            other = 1 - b

            @pl.when(nxt < CPW)
            def _():
                start(nxt, other)

            pltpu.make_async_copy(h2_hbm.at[pl.ds((c0 + cur) * CH, CH)],
                                  rs[slot], ss[slot]).wait()
            pltpu.sync_copy(rs[slot], s_sh.at[idx_v.at[cur]], add=True)
        return carry

    lax.fori_loop(0, CPW // 2, lambda jj, c: pair(jj * 2, c), 0)
    plsc.subcore_barrier()

    pltpu.sync_copy(s_sh.at[pl.ds(lo, rows_per_tile)],
                    s2_out.at[cid, pl.ds(lo, rows_per_tile)])


def _row(b):
    return b.reshape(1, -1).astype(f32)


def kernel(x, s, q, node_pos, senders, receivers, params):
    P = x.shape[1]
    Q = q.shape[1]
    x2, s2, q2 = x[0].astype(f32), s[0].astype(f32), q[0].astype(f32)
    np32 = node_pos.astype(f32)
    r32 = receivers.astype(jnp.int32)
    s32 = senders.astype(jnp.int32)

    npm = jnp.concatenate([2.0 * np32.T,
                           -jnp.sum(np32 * np32, axis=1)[None, :],
                           jnp.zeros((4, N), f32)], axis=0)
    npm_pad = jnp.zeros((8, ND - N), f32).at[3, :].set(-1e30)
    npm = jnp.concatenate([npm, npm_pad], axis=1)
    xa = jnp.concatenate([x2, jnp.ones((P, 1), f32), jnp.zeros((P, 4), f32)], axis=1)
    qa = jnp.concatenate([q2, jnp.ones((Q, 1), f32), jnp.zeros((Q, 4), f32)], axis=1)

    pad = EP - E
    r_g = jnp.concatenate([r32, jnp.zeros((pad,), jnp.int32)]).reshape(EP // CH, CH)
    s_g = jnp.concatenate([s32, jnp.zeros((pad,), jnp.int32)]).reshape(EP // CH, CH)
    r_sc = jnp.concatenate([r32, jnp.full((pad,), N, jnp.int32)]).reshape(EP // CH, CH)

    zeros_nd = jnp.zeros((ND, H), f32)

    enc = params["enc"]
    dec = params["dec"]

    emb = pl.pallas_call(
        _enc_body,
        out_shape=jax.ShapeDtypeStruct((P, H), f32),
    )(s2, enc[0][0], _row(enc[0][1]), enc[1][0], _row(enc[1][1]),
      enc[2][0], _row(enc[2][1]))

    latents = pl.pallas_call(
        _front_body,
        grid=(2, NT),
        in_specs=[
            pl.BlockSpec((P, 8), lambda ph, t: (0, 0)),
            pl.BlockSpec((8, TN), lambda ph, t: (0, t)),
            pl.BlockSpec((P, H), lambda ph, t: (0, 0)),
        ],
        out_specs=pl.BlockSpec((TN, H), lambda ph, t: (t, 0)),
        out_shape=jax.ShapeDtypeStruct((ND, H), f32),
        scratch_shapes=[pltpu.VMEM((P, 1), f32), pltpu.VMEM((P, 1), f32)],
    )(xa, npm, emb)

    mesh = plsc.VectorSubcoreMesh(core_axis_name="c", subcore_axis_name="s",
                                  num_cores=NC, num_subcores=NS)

    gather_call = pl.kernel(
        _gather_body,
        out_type=jax.ShapeDtypeStruct((EP, H), f32),
        mesh=mesh,
        scratch_types=[
            pltpu.VMEM((NPAIR, CH), jnp.int32),
            pltpu.VMEM((NPAIR, CH), jnp.int32),
            pltpu.VMEM((CH, H), f32),
            pltpu.VMEM((CH, H), f32),
            pltpu.VMEM((CH, H), f32),
            pltpu.VMEM((CH, H), f32),
            pltpu.SemaphoreType.DMA,
            pltpu.SemaphoreType.DMA,
            pltpu.SemaphoreType.DMA,
            pltpu.SemaphoreType.DMA,
        ],
    )

    scatter_call = pl.kernel(
        _scatter_body,
        out_type=jax.ShapeDtypeStruct((NC, ND, H), f32),
        mesh=mesh,
        scratch_types=[
            pltpu.VMEM_SHARED((ND, H), f32),
            pltpu.VMEM((CPW, CH), jnp.int32),
            pltpu.VMEM((CH, H), f32),
            pltpu.VMEM((CH, H), f32),
            pltpu.SemaphoreType.DMA,
            pltpu.SemaphoreType.DMA,
        ],
    )

    counts_parts = scatter_call(jnp.ones((EP, H), f32), r_sc, zeros_nd)
    counts2 = counts_parts[:, :, :1]
    for bi, bp in enumerate(params["blocks"]):
        (w1, b1), (w2, b2), (w3, b3) = bp["msg"]
        (nw1, nb1), (nw2, nb2), (nw3, nb3) = bp["node"]
        w1a, w1b = w1[:H], w1[H:]
        nw1a, nw1b = nw1[:H], nw1[H:]

        a_nodes, b_nodes = pl.pallas_call(
            _ab_body,
            grid=(NT,),
            in_specs=[
                pl.BlockSpec((TN, H), lambda t: (t, 0)),
                pl.BlockSpec((H, H), lambda t: (0, 0)),
                pl.BlockSpec((H, H), lambda t: (0, 0)),
                pl.BlockSpec((1, H), lambda t: (0, 0)),
            ],
            out_specs=[pl.BlockSpec((TN, H), lambda t: (t, 0)),
                       pl.BlockSpec((TN, H), lambda t: (t, 0))],
            out_shape=[jax.ShapeDtypeStruct((ND, H), f32),
                       jax.ShapeDtypeStruct((ND, H), f32)],
        )(latents, w1a, w1b, _row(b1))

        g_edges = gather_call(a_nodes, b_nodes, r_g, s_g)

        h2 = pl.pallas_call(
            _h2_body,
            grid=(EP // TE,),
            in_specs=[
                pl.BlockSpec((TE, H), lambda t: (t, 0)),
                pl.BlockSpec((H, H), lambda t: (0, 0)),
                pl.BlockSpec((1, H), lambda t: (0, 0)),
            ],
            out_specs=pl.BlockSpec((TE, H), lambda t: (t, 0)),
            out_shape=jax.ShapeDtypeStruct((EP, H), f32),
        )(g_edges, w2, _row(b2))

        s2_parts = scatter_call(h2, r_sc, zeros_nd)

        latents = pl.pallas_call(
            _node_body,
            grid=(NT,),
            in_specs=[
                pl.BlockSpec((NC, TN, H), lambda t: (0, t, 0)),
                pl.BlockSpec((NC, TN, 1), lambda t: (0, t, 0)),
                pl.BlockSpec((TN, H), lambda t: (t, 0)),
                pl.BlockSpec((H, H), lambda t: (0, 0)),
                pl.BlockSpec((1, H), lambda t: (0, 0)),
                pl.BlockSpec((H, H), lambda t: (0, 0)),
                pl.BlockSpec((H, H), lambda t: (0, 0)),
                pl.BlockSpec((1, H), lambda t: (0, 0)),
                pl.BlockSpec((H, H), lambda t: (0, 0)),
                pl.BlockSpec((1, H), lambda t: (0, 0)),
                pl.BlockSpec((H, H), lambda t: (0, 0)),
                pl.BlockSpec((1, H), lambda t: (0, 0)),
            ],
            out_specs=pl.BlockSpec((TN, H), lambda t: (t, 0)),
            out_shape=jax.ShapeDtypeStruct((ND, H), f32),
        )(s2_parts, counts2, latents,
          w3, _row(b3), nw1a, nw1b, _row(nb1), nw2, _row(nb2), nw3, _row(nb3))

    d1 = dec[0][0]
    d1z = d1[:H]
    d1q = jnp.concatenate([d1[H:], jnp.zeros((5, H), f32)], axis=0)

    out = pl.pallas_call(
        _back_body,
        grid=(2, NT),
        in_specs=[
            pl.BlockSpec((Q, 8), lambda ph, t: (0, 0)),
            pl.BlockSpec((8, TN), lambda ph, t: (0, t)),
            pl.BlockSpec((TN, H), lambda ph, t: (t, 0)),
            pl.BlockSpec((H, H), lambda ph, t: (0, 0)),
            pl.BlockSpec((8, H), lambda ph, t: (0, 0)),
            pl.BlockSpec((1, H), lambda ph, t: (0, 0)),
            pl.BlockSpec((H, H), lambda ph, t: (0, 0)),
            pl.BlockSpec((1, H), lambda ph, t: (0, 0)),
            pl.BlockSpec((H, 8), lambda ph, t: (0, 0)),
            pl.BlockSpec((1, 8), lambda ph, t: (0, 0)),
        ],
        out_specs=pl.BlockSpec((Q, 8), lambda ph, t: (0, 0)),
        out_shape=jax.ShapeDtypeStruct((Q, 8), f32),
        scratch_shapes=[pltpu.VMEM((Q, 1), f32), pltpu.VMEM((Q, 1), f32),
                        pltpu.VMEM((Q, H), f32)],
    )(qa, npm, latents, d1z, d1q, _row(dec[0][1]), dec[1][0], _row(dec[1][1]),
      dec[2][0], _row(dec[2][1]))

    return out[None]


# speedup vs baseline: 1.1201x; 1.0821x over previous
import functools

import jax
import jax.numpy as jnp
from jax import lax
from jax.experimental import pallas as pl
from jax.experimental.pallas import tpu as pltpu
from jax.experimental.pallas import tpu_sc as plsc

N = 10000
E = 320000
H = 128
NC, NS, L = 2, 16, 16
NW = NC * NS
CH = 128
CPW = 80
EP = NW * CPW * CH
ND = 10240
TN = 2048
NT = ND // TN
TE = 4096
PREC = lax.Precision.HIGHEST

f32 = jnp.float32


def _dot(a, b):
    return lax.dot_general(a, b, (((a.ndim - 1,), (0,)), ((), ())),
                           preferred_element_type=f32, precision=PREC)


def _dot_t(a, b):
    return lax.dot_general(a, b, (((0,), (0,)), ((), ())),
                           preferred_element_type=f32, precision=PREC)


def _enc_body(s_ref, w0, b0, w1, b1, w2, b2, out_ref):
    h = jnp.maximum(_dot(s_ref[...], w0[...]) + b0[...], 0.0)
    h = jnp.maximum(_dot(h, w1[...]) + b1[...], 0.0)
    out_ref[...] = _dot(h, w2[...]) + b2[...]


def _front_body(xa_ref, npm_ref, emb_ref, out_ref, m_ref, z_ref):
    ph = pl.program_id(0)
    t = pl.program_id(1)
    logits = _dot(xa_ref[...], npm_ref[...])

    @pl.when(ph == 0)
    def _():
        tm = jnp.max(logits, axis=1, keepdims=True)

        @pl.when(t == 0)
        def _():
            m_ref[...] = tm
            z_ref[...] = jnp.sum(jnp.exp(logits - tm), axis=1, keepdims=True)

        @pl.when(t > 0)
        def _():
            m_old = m_ref[...]
            m_new = jnp.maximum(m_old, tm)
            z_ref[...] = (z_ref[...] * jnp.exp(m_old - m_new)
                          + jnp.sum(jnp.exp(logits - m_new), axis=1, keepdims=True))
            m_ref[...] = m_new

    @pl.when(ph == 1)
    def _():
        w = jnp.exp(logits - m_ref[...]) / z_ref[...]
        out_ref[...] = _dot_t(w, emb_ref[...])


def _ab_body(lat_ref, w1a, w1b, b1, a_ref, b_ref):
    latv = lat_ref[...]
    a_ref[...] = _dot(latv, w1a[...]) + b1[...]
    b_ref[...] = _dot(latv, w1b[...])


def _h2_body(g_ref, w2, b2, out_ref):
    gb = g_ref[...].astype(jnp.bfloat16)
    wb = w2[...].astype(jnp.bfloat16)
    h = lax.dot_general(gb, wb, (((1,), (0,)), ((), ())),
                        preferred_element_type=f32)
    out_ref[...] = jnp.maximum(h + b2[...], 0.0)


def _node_body(s2_ref, c2_ref, lat_ref, w3, b3, nw1a, nw1b, nb1, nw2, nb2,
               nw3, nb3, out_ref):
    s_sum = s2_ref[0] + s2_ref[1]
    counts = c2_ref[0] + c2_ref[1]
    inbox = _dot(s_sum, w3[...]) + counts * b3[...]
    latv = lat_ref[...]
    u = jnp.maximum(_dot(latv, nw1a[...]) + _dot(inbox, nw1b[...]) + nb1[...], 0.0)
    u = jnp.maximum(_dot(u, nw2[...]) + nb2[...], 0.0)
    out_ref[...] = latv + _dot(u, nw3[...]) + nb3[...]


def _back_body(qa_ref, npm_ref, lat_ref, d1z, d1q, db1, d2w, db2, d3w, db3,
               out_ref, m_ref, z_ref, zacc_ref):
    ph = pl.program_id(0)
    t = pl.program_id(1)
    logits = _dot(qa_ref[...], npm_ref[...])

    @pl.when(ph == 0)
    def _():
        tm = jnp.max(logits, axis=1, keepdims=True)

        @pl.when(t == 0)
        def _():
            m_ref[...] = tm
            z_ref[...] = jnp.sum(jnp.exp(logits - tm), axis=1, keepdims=True)

        @pl.when(t > 0)
        def _():
            m_old = m_ref[...]
            m_new = jnp.maximum(m_old, tm)
            z_ref[...] = (z_ref[...] * jnp.exp(m_old - m_new)
                          + jnp.sum(jnp.exp(logits - m_new), axis=1, keepdims=True))
            m_ref[...] = m_new

    @pl.when(ph == 1)
    def _():
        w = jnp.exp(logits - m_ref[...]) / z_ref[...]
        contrib = _dot(w, lat_ref[...])

        @pl.when(t == 0)
        def _():
            zacc_ref[...] = contrib

        @pl.when(t > 0)
        def _():
            zacc_ref[...] = zacc_ref[...] + contrib

        @pl.when(t == NT - 1)
        def _():
            z = zacc_ref[...]
            h = jnp.maximum(_dot(z, d1z[...]) + _dot(qa_ref[...], d1q[...])
                            + db1[...], 0.0)
            h = jnp.maximum(_dot(h, d2w[...]) + db2[...], 0.0)
            out_ref[...] = _dot(h, d3w[...]) + db3[...]


X0 = 118
NPAIR = 160


X0 = 118
NPAIR = 160
HW = H // 2


X0 = 118
NPAIR = 160


def _gather_body(a_hbm, b_hbm, ridx_hbm, sidx_hbm, g_hbm,
                 idxr_v, idxs_v, ra0, rb0, ra1, rb1, sa0, sb0, sa1, sb1):
    cid = lax.axis_index("c")
    sid = lax.axis_index("s")
    base_pair = sid * NPAIR
    pltpu.sync_copy(ridx_hbm.at[pl.ds(base_pair, NPAIR)], idxr_v)
    pltpu.sync_copy(sidx_hbm.at[pl.ds(base_pair, NPAIR)], idxs_v)

    loc0 = cid * X0
    count = jnp.where(cid == 0, X0, NPAIR - X0)

    ras, rbs = (ra0, ra1), (rb0, rb1)
    sas, sbs = (sa0, sa1), (sb0, sb1)

    def start(c_local, slot):
        pltpu.async_copy(a_hbm.at[idxr_v.at[loc0 + c_local]], ras[slot],
                         sas[slot])
        pltpu.async_copy(b_hbm.at[idxs_v.at[loc0 + c_local]], rbs[slot],
                         sbs[slot])

    start(0, 0)

    def pair(j, carry):
        for b in range(2):
            cur = 2 * j + b
            nxt = cur + 1
            slot = b
            other = 1 - b

            @pl.when(nxt < count)
            def _():
                start(nxt, other)

            pltpu.make_async_copy(a_hbm.at[idxr_v.at[loc0 + cur]], ras[slot],
                                  sas[slot]).wait()
            pltpu.make_async_copy(b_hbm.at[idxs_v.at[loc0 + cur]], rbs[slot],
                                  sbs[slot]).wait()
            ra, rb = ras[slot], rbs[slot]

            def row(i, c2):
                for c in range(H // L):
                    sl = pl.ds(c * L, L)
                    ra[i, sl] = jnp.maximum(ra[i, sl] + rb[i, sl], 0.0)
                return c2

            lax.fori_loop(0, CH, row, 0)
            pltpu.sync_copy(ra,
                            g_hbm.at[pl.ds((base_pair + loc0 + cur) * CH, CH)])
        return carry

    lax.fori_loop(0, count // 2, pair, 0)


def _scatter_body(h2_hbm, ridx_hbm, zeros_hbm, s2_out, s_sh,
                  idx_v, r0, r1, s0, s1):
    cid = lax.axis_index("c")
    sid = lax.axis_index("s")
    wid = sid * NC + cid
    c0 = wid * CPW
    rows_per_tile = ND // NS
    lo = sid * rows_per_tile

    pltpu.sync_copy(ridx_hbm.at[pl.ds(c0, CPW)], idx_v)
    pltpu.sync_copy(zeros_hbm.at[pl.ds(lo, rows_per_tile)],
                    s_sh.at[pl.ds(lo, rows_per_tile)])
    plsc.subcore_barrier()

    rs = (r0, r1)
    ss = (s0, s1)

    def start(c_local, slot):
        pltpu.async_copy(h2_hbm.at[pl.ds((c0 + c_local) * CH, CH)],
                         rs[slot], ss[slot])

    start(0, 0)

    def pair(j, carry):
        for b in range(2):
            cur = j + b
            nxt = cur + 1
            slot = b
            other = 1 - b

            @pl.when(nxt < CPW)
            def _():
                start(nxt, other)

            pltpu.make_async_copy(h2_hbm.at[pl.ds((c0 + cur) * CH, CH)],
                                  rs[slot], ss[slot]).wait()
            pltpu.sync_copy(rs[slot], s_sh.at[idx_v.at[cur]], add=True)
        return carry

    lax.fori_loop(0, CPW // 2, lambda jj, c: pair(jj * 2, c), 0)
    plsc.subcore_barrier()

    pltpu.sync_copy(s_sh.at[pl.ds(lo, rows_per_tile)],
                    s2_out.at[cid, pl.ds(lo, rows_per_tile)])


def _row(b):
    return b.reshape(1, -1).astype(f32)


def kernel(x, s, q, node_pos, senders, receivers, params):
    P = x.shape[1]
    Q = q.shape[1]
    x2, s2, q2 = x[0].astype(f32), s[0].astype(f32), q[0].astype(f32)
    np32 = node_pos.astype(f32)
    r32 = receivers.astype(jnp.int32)
    s32 = senders.astype(jnp.int32)

    npm = jnp.concatenate([2.0 * np32.T,
                           -jnp.sum(np32 * np32, axis=1)[None, :],
                           jnp.zeros((4, N), f32)], axis=0)
    npm_pad = jnp.zeros((8, ND - N), f32).at[3, :].set(-1e30)
    npm = jnp.concatenate([npm, npm_pad], axis=1)
    xa = jnp.concatenate([x2, jnp.ones((P, 1), f32), jnp.zeros((P, 4), f32)], axis=1)
    qa = jnp.concatenate([q2, jnp.ones((Q, 1), f32), jnp.zeros((Q, 4), f32)], axis=1)

    pad = EP - E
    r_g = jnp.concatenate([r32, jnp.zeros((pad,), jnp.int32)]).reshape(EP // CH, CH)
    s_g = jnp.concatenate([s32, jnp.zeros((pad,), jnp.int32)]).reshape(EP // CH, CH)
    r_sc = jnp.concatenate([r32, jnp.full((pad,), N, jnp.int32)]).reshape(EP // CH, CH)

    zeros_nd = jnp.zeros((ND, H), f32)

    enc = params["enc"]
    dec = params["dec"]

    mesh = plsc.VectorSubcoreMesh(core_axis_name="c", subcore_axis_name="s",
                                  num_cores=NC, num_subcores=NS)

    gather_call = pl.kernel(
        _gather_body,
        out_type=jax.ShapeDtypeStruct((EP, H), f32),
        mesh=mesh,
        scratch_types=[
            pltpu.VMEM((NPAIR, CH), jnp.int32),
            pltpu.VMEM((NPAIR, CH), jnp.int32),
            pltpu.VMEM((CH, H), f32),
            pltpu.VMEM((CH, H), f32),
            pltpu.VMEM((CH, H), f32),
            pltpu.VMEM((CH, H), f32),
            pltpu.SemaphoreType.DMA,
            pltpu.SemaphoreType.DMA,
            pltpu.SemaphoreType.DMA,
            pltpu.SemaphoreType.DMA,
        ],
    )

    scatter_call = pl.kernel(
        _scatter_body,
        out_type=jax.ShapeDtypeStruct((NC, ND, H), f32),
        mesh=mesh,
        scratch_types=[
            pltpu.VMEM_SHARED((ND, H), f32),
            pltpu.VMEM((CPW, CH), jnp.int32),
            pltpu.VMEM((CH, H), f32),
            pltpu.VMEM((CH, H), f32),
            pltpu.SemaphoreType.DMA,
            pltpu.SemaphoreType.DMA,
        ],
    )

    counts_parts = scatter_call(jnp.ones((EP, H), f32), r_sc, zeros_nd)
    counts2 = counts_parts[:, :, :1]

    emb = pl.pallas_call(
        _enc_body,
        out_shape=jax.ShapeDtypeStruct((P, H), f32),
    )(s2, enc[0][0], _row(enc[0][1]), enc[1][0], _row(enc[1][1]),
      enc[2][0], _row(enc[2][1]))

    latents = pl.pallas_call(
        _front_body,
        grid=(2, NT),
        in_specs=[
            pl.BlockSpec((P, 8), lambda ph, t: (0, 0)),
            pl.BlockSpec((8, TN), lambda ph, t: (0, t)),
            pl.BlockSpec((P, H), lambda ph, t: (0, 0)),
        ],
        out_specs=pl.BlockSpec((TN, H), lambda ph, t: (t, 0)),
        out_shape=jax.ShapeDtypeStruct((ND, H), f32),
        scratch_shapes=[pltpu.VMEM((P, 1), f32), pltpu.VMEM((P, 1), f32)],
    )(xa, npm, emb)

    for bi, bp in enumerate(params["blocks"]):
        (w1, b1), (w2, b2), (w3, b3) = bp["msg"]
        (nw1, nb1), (nw2, nb2), (nw3, nb3) = bp["node"]
        w1a, w1b = w1[:H], w1[H:]
        nw1a, nw1b = nw1[:H], nw1[H:]

        a_nodes, b_nodes = pl.pallas_call(
            _ab_body,
            grid=(NT,),
            in_specs=[
                pl.BlockSpec((TN, H), lambda t: (t, 0)),
                pl.BlockSpec((H, H), lambda t: (0, 0)),
                pl.BlockSpec((H, H), lambda t: (0, 0)),
                pl.BlockSpec((1, H), lambda t: (0, 0)),
            ],
            out_specs=[pl.BlockSpec((TN, H), lambda t: (t, 0)),
                       pl.BlockSpec((TN, H), lambda t: (t, 0))],
            out_shape=[jax.ShapeDtypeStruct((ND, H), f32),
                       jax.ShapeDtypeStruct((ND, H), f32)],
        )(latents, w1a, w1b, _row(b1))

        g_edges = gather_call(a_nodes, b_nodes, r_g, s_g)

        h2 = pl.pallas_call(
            _h2_body,
            grid=(EP // TE,),
            in_specs=[
                pl.BlockSpec((TE, H), lambda t: (t, 0)),
                pl.BlockSpec((H, H), lambda t: (0, 0)),
                pl.BlockSpec((1, H), lambda t: (0, 0)),
            ],
            out_specs=pl.BlockSpec((TE, H), lambda t: (t, 0)),
            out_shape=jax.ShapeDtypeStruct((EP, H), f32),
        )(g_edges, w2, _row(b2))

        s2_parts = scatter_call(h2, r_sc, zeros_nd)

        latents = pl.pallas_call(
            _node_body,
            grid=(NT,),
            in_specs=[
                pl.BlockSpec((NC, TN, H), lambda t: (0, t, 0)),
                pl.BlockSpec((NC, TN, 1), lambda t: (0, t, 0)),
                pl.BlockSpec((TN, H), lambda t: (t, 0)),
                pl.BlockSpec((H, H), lambda t: (0, 0)),
                pl.BlockSpec((1, H), lambda t: (0, 0)),
                pl.BlockSpec((H, H), lambda t: (0, 0)),
                pl.BlockSpec((H, H), lambda t: (0, 0)),
                pl.BlockSpec((1, H), lambda t: (0, 0)),
                pl.BlockSpec((H, H), lambda t: (0, 0)),
                pl.BlockSpec((1, H), lambda t: (0, 0)),
                pl.BlockSpec((H, H), lambda t: (0, 0)),
                pl.BlockSpec((1, H), lambda t: (0, 0)),
            ],
            out_specs=pl.BlockSpec((TN, H), lambda t: (t, 0)),
            out_shape=jax.ShapeDtypeStruct((ND, H), f32),
        )(s2_parts, counts2, latents,
          w3, _row(b3), nw1a, nw1b, _row(nb1), nw2, _row(nb2), nw3, _row(nb3))

    d1 = dec[0][0]
    d1z = d1[:H]
    d1q = jnp.concatenate([d1[H:], jnp.zeros((5, H), f32)], axis=0)

    out = pl.pallas_call(
        _back_body,
        grid=(2, NT),
        in_specs=[
            pl.BlockSpec((Q, 8), lambda ph, t: (0, 0)),
            pl.BlockSpec((8, TN), lambda ph, t: (0, t)),
            pl.BlockSpec((TN, H), lambda ph, t: (t, 0)),
            pl.BlockSpec((H, H), lambda ph, t: (0, 0)),
            pl.BlockSpec((8, H), lambda ph, t: (0, 0)),
            pl.BlockSpec((1, H), lambda ph, t: (0, 0)),
            pl.BlockSpec((H, H), lambda ph, t: (0, 0)),
            pl.BlockSpec((1, H), lambda ph, t: (0, 0)),
            pl.BlockSpec((H, 8), lambda ph, t: (0, 0)),
            pl.BlockSpec((1, 8), lambda ph, t: (0, 0)),
        ],
        out_specs=pl.BlockSpec((Q, 8), lambda ph, t: (0, 0)),
        out_shape=jax.ShapeDtypeStruct((Q, 8), f32),
        scratch_shapes=[pltpu.VMEM((Q, 1), f32), pltpu.VMEM((Q, 1), f32),
                        pltpu.VMEM((Q, H), f32)],
    )(qa, npm, latents, d1z, d1q, _row(dec[0][1]), dec[1][0], _row(dec[1][1]),
      dec[2][0], _row(dec[2][1]))

    return out[None]


# speedup vs baseline: 1.2945x; 1.1557x over previous
import functools

import jax
import jax.numpy as jnp
from jax import lax
from jax.experimental import pallas as pl
from jax.experimental.pallas import tpu as pltpu
from jax.experimental.pallas import tpu_sc as plsc

N = 10000
E = 320000
H = 128
NC, NS, L = 2, 16, 16
NW = NC * NS
CH = 128
CPW = 80
EP = NW * CPW * CH
ND = 10240
TN = 2048
NT = ND // TN
TE = 4096
PREC = lax.Precision.HIGHEST

f32 = jnp.float32


def _dot(a, b):
    return lax.dot_general(a, b, (((a.ndim - 1,), (0,)), ((), ())),
                           preferred_element_type=f32, precision=PREC)


def _dot_t(a, b):
    return lax.dot_general(a, b, (((0,), (0,)), ((), ())),
                           preferred_element_type=f32, precision=PREC)


def _enc_body(s_ref, w0, b0, w1, b1, w2, b2, out_ref):
    h = jnp.maximum(_dot(s_ref[...], w0[...]) + b0[...], 0.0)
    h = jnp.maximum(_dot(h, w1[...]) + b1[...], 0.0)
    out_ref[...] = _dot(h, w2[...]) + b2[...]


def _front_body(xa_ref, npm_ref, emb_ref, out_ref, z_ref):
    ph = pl.program_id(0)
    t = pl.program_id(1)
    logits = _dot(xa_ref[...], npm_ref[...])
    w = jnp.exp(logits)

    @pl.when(ph == 0)
    def _():
        zt = jnp.sum(w, axis=1, keepdims=True)

        @pl.when(t == 0)
        def _():
            z_ref[...] = zt

        @pl.when(t > 0)
        def _():
            z_ref[...] = z_ref[...] + zt

    @pl.when(ph == 1)
    def _():
        wn = (w / z_ref[...]).astype(jnp.bfloat16)
        out_ref[...] = lax.dot_general(
            wn, emb_ref[...].astype(jnp.bfloat16), (((0,), (0,)), ((), ())),
            preferred_element_type=f32)


def _ab_body(lat_ref, w1a, w1b, b1, a_ref, b_ref):
    latv = lat_ref[...]
    a_ref[...] = _dot(latv, w1a[...]) + b1[...]
    b_ref[...] = _dot(latv, w1b[...])


def _h2_body(g_ref, w2, b2, out_ref):
    gb = g_ref[...].astype(jnp.bfloat16)
    wb = w2[...].astype(jnp.bfloat16)
    h = lax.dot_general(gb, wb, (((1,), (0,)), ((), ())),
                        preferred_element_type=f32)
    out_ref[...] = jnp.maximum(h + b2[...], 0.0)


def _node_body(s2_ref, c2_ref, lat_ref, w3, b3, nw1a, nw1b, nb1, nw2, nb2,
               nw3, nb3, out_ref):
    s_sum = s2_ref[0] + s2_ref[1]
    counts = c2_ref[0] + c2_ref[1]
    inbox = _dot(s_sum, w3[...]) + counts * b3[...]
    latv = lat_ref[...]
    u = jnp.maximum(_dot(latv, nw1a[...]) + _dot(inbox, nw1b[...]) + nb1[...], 0.0)
    u = jnp.maximum(_dot(u, nw2[...]) + nb2[...], 0.0)
    out_ref[...] = latv + _dot(u, nw3[...]) + nb3[...]


def _back_body(qa_ref, npm_ref, lat_ref, d1z, d1q, db1, d2w, db2, d3w, db3,
               out_ref, z_ref, zacc_ref):
    ph = pl.program_id(0)
    t = pl.program_id(1)
    logits = _dot(qa_ref[...], npm_ref[...])
    w = jnp.exp(logits)

    @pl.when(ph == 0)
    def _():
        zt = jnp.sum(w, axis=1, keepdims=True)

        @pl.when(t == 0)
        def _():
            z_ref[...] = zt

        @pl.when(t > 0)
        def _():
            z_ref[...] = z_ref[...] + zt

    @pl.when(ph == 1)
    def _():
        contrib = lax.dot_general(
            w.astype(jnp.bfloat16), lat_ref[...].astype(jnp.bfloat16),
            (((1,), (0,)), ((), ())), preferred_element_type=f32)

        @pl.when(t == 0)
        def _():
            zacc_ref[...] = contrib

        @pl.when(t > 0)
        def _():
            zacc_ref[...] = zacc_ref[...] + contrib

        @pl.when(t == NT - 1)
        def _():
            z = zacc_ref[...] / z_ref[...]
            h = jnp.maximum(_dot(z, d1z[...]) + _dot(qa_ref[...], d1q[...])
                            + db1[...], 0.0)
            h = jnp.maximum(_dot(h, d2w[...]) + db2[...], 0.0)
            out_ref[...] = _dot(h, d3w[...]) + db3[...]


X0 = 118
NPAIR = 160


X0 = 118
NPAIR = 160
HW = H // 2


X0 = 118
NPAIR = 160


def _gather_body(a_hbm, b_hbm, ridx_hbm, sidx_hbm, g_hbm,
                 idxr_v, idxs_v, ra0, rb0, ra1, rb1, sa0, sb0, sa1, sb1):
    cid = lax.axis_index("c")
    sid = lax.axis_index("s")
    base_pair = sid * NPAIR
    pltpu.sync_copy(ridx_hbm.at[pl.ds(base_pair, NPAIR)], idxr_v)
    pltpu.sync_copy(sidx_hbm.at[pl.ds(base_pair, NPAIR)], idxs_v)

    loc0 = cid * X0
    count = jnp.where(cid == 0, X0, NPAIR - X0)

    ras, rbs = (ra0, ra1), (rb0, rb1)
    sas, sbs = (sa0, sa1), (sb0, sb1)

    def start(c_local, slot):
        pltpu.async_copy(a_hbm.at[idxr_v.at[loc0 + c_local]], ras[slot],
                         sas[slot])
        pltpu.async_copy(b_hbm.at[idxs_v.at[loc0 + c_local]], rbs[slot],
                         sbs[slot])

    start(0, 0)

    def pair(j, carry):
        for b in range(2):
            cur = 2 * j + b
            nxt = cur + 1
            slot = b
            other = 1 - b

            @pl.when(nxt < count)
            def _():
                start(nxt, other)

            pltpu.make_async_copy(a_hbm.at[idxr_v.at[loc0 + cur]], ras[slot],
                                  sas[slot]).wait()
            pltpu.make_async_copy(b_hbm.at[idxs_v.at[loc0 + cur]], rbs[slot],
                                  sbs[slot]).wait()
            ra, rb = ras[slot], rbs[slot]

            def row(i, c2):
                for c in range(H // L):
                    sl = pl.ds(c * L, L)
                    ra[i, sl] = jnp.maximum(ra[i, sl] + rb[i, sl], 0.0)
                return c2

            lax.fori_loop(0, CH, row, 0)
            pltpu.sync_copy(ra,
                            g_hbm.at[pl.ds((base_pair + loc0 + cur) * CH, CH)])
        return carry

    lax.fori_loop(0, count // 2, pair, 0)


def _scatter_body(h2_hbm, ridx_hbm, zeros_hbm, s2_out, s_sh,
                  idx_v, r0, r1, s0, s1):
    cid = lax.axis_index("c")
    sid = lax.axis_index("s")
    wid = sid * NC + cid
    c0 = wid * CPW
    rows_per_tile = ND // NS
    lo = sid * rows_per_tile

    pltpu.sync_copy(ridx_hbm.at[pl.ds(c0, CPW)], idx_v)
    pltpu.sync_copy(zeros_hbm.at[pl.ds(lo, rows_per_tile)],
                    s_sh.at[pl.ds(lo, rows_per_tile)])
    plsc.subcore_barrier()

    rs = (r0, r1)
    ss = (s0, s1)

    def start(c_local, slot):
        pltpu.async_copy(h2_hbm.at[pl.ds((c0 + c_local) * CH, CH)],
                         rs[slot], ss[slot])

    start(0, 0)

    def pair(j, carry):
        for b in range(2):
            cur = j + b
            nxt = cur + 1
            slot = b
            other = 1 - b

            @pl.when(nxt < CPW)
            def _():
                start(nxt, other)

            pltpu.make_async_copy(h2_hbm.at[pl.ds((c0 + cur) * CH, CH)],
                                  rs[slot], ss[slot]).wait()
            pltpu.sync_copy(rs[slot], s_sh.at[idx_v.at[cur]], add=True)
        return carry

    lax.fori_loop(0, CPW // 2, lambda jj, c: pair(jj * 2, c), 0)
    plsc.subcore_barrier()

    pltpu.sync_copy(s_sh.at[pl.ds(lo, rows_per_tile)],
                    s2_out.at[cid, pl.ds(lo, rows_per_tile)])


def _counts_body(ridx_hbm, zeros_hbm, ones_hbm, c_out, c_sh, idx_v, ones_v):
    cid = lax.axis_index("c")
    sid = lax.axis_index("s")
    wid = sid * NC + cid
    c0 = wid * CPW
    rows_per_tile = ND // NS
    lo = sid * rows_per_tile

    pltpu.sync_copy(ridx_hbm.at[pl.ds(c0, CPW)], idx_v)
    pltpu.sync_copy(ones_hbm, ones_v)
    pltpu.sync_copy(zeros_hbm.at[pl.ds(lo, rows_per_tile)],
                    c_sh.at[pl.ds(lo, rows_per_tile)])
    plsc.subcore_barrier()

    def chunk2(j, carry):
        pltpu.sync_copy(ones_v, c_sh.at[idx_v.at[j]], add=True)
        return carry

    lax.fori_loop(0, CPW, chunk2, 0)
    plsc.subcore_barrier()
    pltpu.sync_copy(c_sh.at[pl.ds(lo, rows_per_tile)],
                    c_out.at[cid, pl.ds(lo, rows_per_tile)])


def _row(b):
    return b.reshape(1, -1).astype(f32)


def kernel(x, s, q, node_pos, senders, receivers, params):
    P = x.shape[1]
    Q = q.shape[1]
    x2, s2, q2 = x[0].astype(f32), s[0].astype(f32), q[0].astype(f32)
    np32 = node_pos.astype(f32)
    r32 = receivers.astype(jnp.int32)
    s32 = senders.astype(jnp.int32)

    npm = jnp.concatenate([2.0 * np32.T,
                           -jnp.sum(np32 * np32, axis=1)[None, :],
                           jnp.zeros((4, N), f32)], axis=0)
    npm_pad = jnp.zeros((8, ND - N), f32).at[3, :].set(-1e30)
    npm = jnp.concatenate([npm, npm_pad], axis=1)
    xa = jnp.concatenate([x2, jnp.ones((P, 1), f32), jnp.zeros((P, 4), f32)], axis=1)
    qa = jnp.concatenate([q2, jnp.ones((Q, 1), f32), jnp.zeros((Q, 4), f32)], axis=1)

    pad = EP - E
    r_g = jnp.concatenate([r32, jnp.zeros((pad,), jnp.int32)]).reshape(EP // CH, CH)
    s_g = jnp.concatenate([s32, jnp.zeros((pad,), jnp.int32)]).reshape(EP // CH, CH)
    r_sc = jnp.concatenate([r32, jnp.full((pad,), N, jnp.int32)]).reshape(EP // CH, CH)

    zeros_nd = jnp.zeros((ND, H), f32)

    enc = params["enc"]
    dec = params["dec"]

    mesh = plsc.VectorSubcoreMesh(core_axis_name="c", subcore_axis_name="s",
                                  num_cores=NC, num_subcores=NS)

    gather_call = pl.kernel(
        _gather_body,
        out_type=jax.ShapeDtypeStruct((EP, H), f32),
        mesh=mesh,
        scratch_types=[
            pltpu.VMEM((NPAIR, CH), jnp.int32),
            pltpu.VMEM((NPAIR, CH), jnp.int32),
            pltpu.VMEM((CH, H), f32),
            pltpu.VMEM((CH, H), f32),
            pltpu.VMEM((CH, H), f32),
            pltpu.VMEM((CH, H), f32),
            pltpu.SemaphoreType.DMA,
            pltpu.SemaphoreType.DMA,
            pltpu.SemaphoreType.DMA,
            pltpu.SemaphoreType.DMA,
        ],
    )

    scatter_call = pl.kernel(
        _scatter_body,
        out_type=jax.ShapeDtypeStruct((NC, ND, H), f32),
        mesh=mesh,
        scratch_types=[
            pltpu.VMEM_SHARED((ND, H), f32),
            pltpu.VMEM((CPW, CH), jnp.int32),
            pltpu.VMEM((CH, H), f32),
            pltpu.VMEM((CH, H), f32),
            pltpu.SemaphoreType.DMA,
            pltpu.SemaphoreType.DMA,
        ],
    )

    counts_call = pl.kernel(
        _counts_body,
        out_type=jax.ShapeDtypeStruct((NC, ND, H), f32),
        mesh=mesh,
        scratch_types=[
            pltpu.VMEM_SHARED((ND, H), f32),
            pltpu.VMEM((CPW, CH), jnp.int32),
            pltpu.VMEM((CH, H), f32),
        ],
    )
    counts_parts = counts_call(r_sc, zeros_nd, jnp.ones((CH, H), f32))
    counts2 = counts_parts[:, :, :1]

    emb = pl.pallas_call(
        _enc_body,
        out_shape=jax.ShapeDtypeStruct((P, H), f32),
    )(s2, enc[0][0], _row(enc[0][1]), enc[1][0], _row(enc[1][1]),
      enc[2][0], _row(enc[2][1]))

    latents = pl.pallas_call(
        _front_body,
        grid=(2, NT),
        in_specs=[
            pl.BlockSpec((P, 8), lambda ph, t: (0, 0)),
            pl.BlockSpec((8, TN), lambda ph, t: (0, t)),
            pl.BlockSpec((P, H), lambda ph, t: (0, 0)),
        ],
        out_specs=pl.BlockSpec((TN, H), lambda ph, t: (t, 0)),
        out_shape=jax.ShapeDtypeStruct((ND, H), f32),
        scratch_shapes=[pltpu.VMEM((P, 1), f32)],
    )(xa, npm, emb)

    for bi, bp in enumerate(params["blocks"]):
        (w1, b1), (w2, b2), (w3, b3) = bp["msg"]
        (nw1, nb1), (nw2, nb2), (nw3, nb3) = bp["node"]
        w1a, w1b = w1[:H], w1[H:]
        nw1a, nw1b = nw1[:H], nw1[H:]

        a_nodes, b_nodes = pl.pallas_call(
            _ab_body,
            grid=(NT,),
            in_specs=[
                pl.BlockSpec((TN, H), lambda t: (t, 0)),
                pl.BlockSpec((H, H), lambda t: (0, 0)),
                pl.BlockSpec((H, H), lambda t: (0, 0)),
                pl.BlockSpec((1, H), lambda t: (0, 0)),
            ],
            out_specs=[pl.BlockSpec((TN, H), lambda t: (t, 0)),
                       pl.BlockSpec((TN, H), lambda t: (t, 0))],
            out_shape=[jax.ShapeDtypeStruct((ND, H), f32),
                       jax.ShapeDtypeStruct((ND, H), f32)],
        )(latents, w1a, w1b, _row(b1))

        g_edges = gather_call(a_nodes, b_nodes, r_g, s_g)

        h2 = pl.pallas_call(
            _h2_body,
            grid=(EP // TE,),
            in_specs=[
                pl.BlockSpec((TE, H), lambda t: (t, 0)),
                pl.BlockSpec((H, H), lambda t: (0, 0)),
                pl.BlockSpec((1, H), lambda t: (0, 0)),
            ],
            out_specs=pl.BlockSpec((TE, H), lambda t: (t, 0)),
            out_shape=jax.ShapeDtypeStruct((EP, H), f32),
        )(g_edges, w2, _row(b2))

        s2_parts = scatter_call(h2, r_sc, zeros_nd)

        latents = pl.pallas_call(
            _node_body,
            grid=(NT,),
            in_specs=[
                pl.BlockSpec((NC, TN, H), lambda t: (0, t, 0)),
                pl.BlockSpec((NC, TN, 1), lambda t: (0, t, 0)),
                pl.BlockSpec((TN, H), lambda t: (t, 0)),
                pl.BlockSpec((H, H), lambda t: (0, 0)),
                pl.BlockSpec((1, H), lambda t: (0, 0)),
                pl.BlockSpec((H, H), lambda t: (0, 0)),
                pl.BlockSpec((H, H), lambda t: (0, 0)),
                pl.BlockSpec((1, H), lambda t: (0, 0)),
                pl.BlockSpec((H, H), lambda t: (0, 0)),
                pl.BlockSpec((1, H), lambda t: (0, 0)),
                pl.BlockSpec((H, H), lambda t: (0, 0)),
                pl.BlockSpec((1, H), lambda t: (0, 0)),
            ],
            out_specs=pl.BlockSpec((TN, H), lambda t: (t, 0)),
            out_shape=jax.ShapeDtypeStruct((ND, H), f32),
        )(s2_parts, counts2, latents,
          w3, _row(b3), nw1a, nw1b, _row(nb1), nw2, _row(nb2), nw3, _row(nb3))

    d1 = dec[0][0]
    d1z = d1[:H]
    d1q = jnp.concatenate([d1[H:], jnp.zeros((5, H), f32)], axis=0)

    out = pl.pallas_call(
        _back_body,
        grid=(2, NT),
        in_specs=[
            pl.BlockSpec((Q, 8), lambda ph, t: (0, 0)),
            pl.BlockSpec((8, TN), lambda ph, t: (0, t)),
            pl.BlockSpec((TN, H), lambda ph, t: (t, 0)),
            pl.BlockSpec((H, H), lambda ph, t: (0, 0)),
            pl.BlockSpec((8, H), lambda ph, t: (0, 0)),
            pl.BlockSpec((1, H), lambda ph, t: (0, 0)),
            pl.BlockSpec((H, H), lambda ph, t: (0, 0)),
            pl.BlockSpec((1, H), lambda ph, t: (0, 0)),
            pl.BlockSpec((H, 8), lambda ph, t: (0, 0)),
            pl.BlockSpec((1, 8), lambda ph, t: (0, 0)),
        ],
        out_specs=pl.BlockSpec((Q, 8), lambda ph, t: (0, 0)),
        out_shape=jax.ShapeDtypeStruct((Q, 8), f32),
        scratch_shapes=[pltpu.VMEM((Q, 1), f32), pltpu.VMEM((Q, H), f32)],
    )(qa, npm, latents, d1z, d1q, _row(dec[0][1]), dec[1][0], _row(dec[1][1]),
      dec[2][0], _row(dec[2][1]))

    return out[None]
